# baseline (speedup 1.0000x reference)
import functools

import jax
import jax.numpy as jnp
from jax import lax
from jax.experimental import pallas as pl
from jax.experimental.pallas import tpu as pltpu
from jax.experimental.pallas import tpu_sc as plsc

_B = 16384
_D = 50
_CHUNK = 256


def _gather_body(nc, b_per_w,
                 user_hbm, item_hbm, utab_hbm, itab_hbm, uout_hbm, iout_hbm,
                 uidx_v, iidx_v, urows_v, irows_v, gsem):
    wid = lax.axis_index("s") * nc + lax.axis_index("c")
    base = wid * b_per_w
    pltpu.sync_copy(user_hbm.at[wid], uidx_v)
    pltpu.sync_copy(item_hbm.at[wid], iidx_v)
    lanes = lax.iota(jnp.int32, 16)
    n_chunks = b_per_w // _CHUNK

    for c in range(n_chunks):
        def group(k, _, c=c):
            off = pl.multiple_of(c * _CHUNK + k * 16, 16)
            uvec = uidx_v[pl.ds(off, 16)]
            ivec = iidx_v[pl.ds(off, 16)]
            for j in range(16):
                us = jnp.sum(jnp.where(lanes == j, uvec, 0))
                isc = jnp.sum(jnp.where(lanes == j, ivec, 0))
                row = k * 16 + j
                pltpu.async_copy(
                    utab_hbm.at[us >> 3].at[pl.ds(us & 7, 1)],
                    urows_v.at[pl.ds(row, 1)], gsem)
                pltpu.async_copy(
                    itab_hbm.at[isc >> 3].at[pl.ds(isc & 7, 1)],
                    irows_v.at[pl.ds(row, 1)], gsem)
            return 0

        lax.fori_loop(0, _CHUNK // 16, group, 0)
        pltpu.make_async_copy(uout_hbm.at[pl.ds(0, _CHUNK)], urows_v,
                              gsem).wait()
        pltpu.make_async_copy(uout_hbm.at[pl.ds(0, _CHUNK)], irows_v,
                              gsem).wait()
        pltpu.sync_copy(urows_v, uout_hbm.at[pl.ds(base + c * _CHUNK, _CHUNK)])
        pltpu.sync_copy(irows_v, iout_hbm.at[pl.ds(base + c * _CHUNK, _CHUNK)])


def _mlp_body(u_ref, i_ref, w1a_ref, w1b_ref, b1_ref, w2_ref, b2_ref,
              w3_ref, b3_ref, out_ref):
    hp = jax.lax.Precision.HIGHEST
    h = jnp.dot(u_ref[...], w1a_ref[...], precision=hp)
    h = h + jnp.dot(i_ref[...], w1b_ref[...], precision=hp)
    h = jnp.maximum(h + b1_ref[...], 0.0)
    h = jnp.maximum(jnp.dot(h, w2_ref[...], precision=hp) + b2_ref[...], 0.0)
    out_ref[...] = jnp.dot(h, w3_ref[...], precision=hp) + b3_ref[...]


def kernel(user, item, user_table, item_table, W1, b1, W2, b2, W3, b3):
    info = plsc.get_sparse_core_info()
    nc, ns = info.num_cores, info.num_subcores
    nw = nc * ns
    b_per_w = _B // nw

    mesh = plsc.VectorSubcoreMesh(core_axis_name="c", subcore_axis_name="s")
    gather = functools.partial(
        pl.kernel,
        out_type=(jax.ShapeDtypeStruct((_B, _D), jnp.float32),
                  jax.ShapeDtypeStruct((_B, _D), jnp.float32)),
        mesh=mesh,
        scratch_types=[
            pltpu.VMEM((b_per_w,), jnp.int32),
            pltpu.VMEM((b_per_w,), jnp.int32),
            pltpu.VMEM((_CHUNK, _D), jnp.float32),
            pltpu.VMEM((_CHUNK, _D), jnp.float32),
            pltpu.SemaphoreType.DMA,
        ],
        compiler_params=pltpu.CompilerParams(needs_layout_passes=False),
    )(functools.partial(_gather_body, nc, b_per_w))

    u_rows, i_rows = gather(
        user.reshape(nw, b_per_w).astype(jnp.int32),
        item.reshape(nw, b_per_w).astype(jnp.int32),
        user_table.reshape(-1, 8, _D), item_table.reshape(-1, 8, _D))

    bb = 2048
    grid = _B // bb
    w1aT = W1[:, :_D].T
    w1bT = W1[:, _D:].T
    out = pl.pallas_call(
        _mlp_body,
        grid=(grid,),
        in_specs=[
            pl.BlockSpec((bb, _D), lambda i: (i, 0)),
            pl.BlockSpec((bb, _D), lambda i: (i, 0)),
            pl.BlockSpec(w1aT.shape, lambda i: (0, 0)),
            pl.BlockSpec(w1bT.shape, lambda i: (0, 0)),
            pl.BlockSpec((1, 128), lambda i: (0, 0)),
            pl.BlockSpec((128, 64), lambda i: (0, 0)),
            pl.BlockSpec((1, 64), lambda i: (0, 0)),
            pl.BlockSpec((64, 1), lambda i: (0, 0)),
            pl.BlockSpec((1, 1), lambda i: (0, 0)),
        ],
        out_specs=pl.BlockSpec((bb, 1), lambda i: (i, 0)),
        out_shape=jax.ShapeDtypeStruct((_B, 1), jnp.float32),
    )(u_rows, i_rows, w1aT, w1bT, b1.reshape(1, 128), W2.T,
      b2.reshape(1, 64), W3.T, b3.reshape(1, 1))
    return out


# speedup vs baseline: 2.4233x; 2.4233x over previous
import functools

import jax
import jax.numpy as jnp
from jax import lax
from jax.experimental import pallas as pl
from jax.experimental.pallas import tpu as pltpu
from jax.experimental.pallas import tpu_sc as plsc

_B = 16384
_D = 50
_CHUNK = 256


def _gather_body(nc, b_per_w,
                 user_hbm, item_hbm, utab_hbm, itab_hbm, uout_hbm, iout_hbm,
                 uidx_v, iidx_v, urows_v, irows_v, gsem):
    wid = lax.axis_index("s") * nc + lax.axis_index("c")
    base = wid * b_per_w
    pltpu.sync_copy(user_hbm.at[wid], uidx_v)
    pltpu.sync_copy(item_hbm.at[wid], iidx_v)
    lanes = lax.iota(jnp.int32, 16)
    n_chunks = b_per_w // _CHUNK

    for c in range(n_chunks):
        def group(k, _, c=c):
            off = pl.multiple_of(c * _CHUNK + k * 16, 16)
            uvec = uidx_v[pl.ds(off, 16)]
            ivec = iidx_v[pl.ds(off, 16)]
            for j in range(16):
                us = jnp.sum(jnp.where(lanes == j, uvec, 0))
                isc = jnp.sum(jnp.where(lanes == j, ivec, 0))
                row = k * 16 + j
                pltpu.async_copy(utab_hbm.at[pl.ds(us, 1)],
                                 urows_v.at[pl.ds(row, 1)], gsem)
                pltpu.async_copy(itab_hbm.at[pl.ds(isc, 1)],
                                 irows_v.at[pl.ds(row, 1)], gsem)
            return 0

        lax.fori_loop(0, _CHUNK // 16, group, 0)
        pltpu.make_async_copy(uout_hbm.at[pl.ds(0, _CHUNK)], urows_v,
                              gsem).wait()
        pltpu.make_async_copy(uout_hbm.at[pl.ds(0, _CHUNK)], irows_v,
                              gsem).wait()
        pltpu.sync_copy(urows_v, uout_hbm.at[pl.ds(base + c * _CHUNK, _CHUNK)])
        pltpu.sync_copy(irows_v, iout_hbm.at[pl.ds(base + c * _CHUNK, _CHUNK)])


def _mlp_body(u_ref, i_ref, w1a_ref, w1b_ref, b1_ref, w2_ref, b2_ref,
              w3_ref, b3_ref, out_ref):
    u = u_ref[...]
    i = i_ref[...]
    h = jnp.dot(u, w1a_ref[...], preferred_element_type=jnp.float32)
    h = h + jnp.dot(i, w1b_ref[...], preferred_element_type=jnp.float32)
    h = jnp.maximum(h + b1_ref[...], 0.0)
    h = jnp.maximum(
        jnp.dot(h, w2_ref[...], preferred_element_type=jnp.float32)
        + b2_ref[...], 0.0)
    out_ref[...] = (jnp.dot(h, w3_ref[...],
                            preferred_element_type=jnp.float32)
                    + b3_ref[...])


def kernel(user, item, user_table, item_table, W1, b1, W2, b2, W3, b3):
    info = plsc.get_sparse_core_info()
    nc, ns = info.num_cores, info.num_subcores
    nw = nc * ns
    b_per_w = _B // nw

    mesh = plsc.VectorSubcoreMesh(core_axis_name="c", subcore_axis_name="s")
    gather = functools.partial(
        pl.kernel,
        out_type=(jax.ShapeDtypeStruct((_B, _D), jnp.float32),
                  jax.ShapeDtypeStruct((_B, _D), jnp.float32)),
        mesh=mesh,
        scratch_types=[
            pltpu.VMEM((b_per_w,), jnp.int32),
            pltpu.VMEM((b_per_w,), jnp.int32),
            pltpu.VMEM((_CHUNK, _D), jnp.float32),
            pltpu.VMEM((_CHUNK, _D), jnp.float32),
            pltpu.SemaphoreType.DMA,
        ],
        compiler_params=pltpu.CompilerParams(needs_layout_passes=False),
    )(functools.partial(_gather_body, nc, b_per_w))

    u_rows, i_rows = gather(
        user.reshape(nw, b_per_w).astype(jnp.int32),
        item.reshape(nw, b_per_w).astype(jnp.int32),
        user_table, item_table)

    bb = 2048
    grid = _B // bb
    w1aT = W1[:, :_D].T
    w1bT = W1[:, _D:].T
    out = pl.pallas_call(
        _mlp_body,
        grid=(grid,),
        in_specs=[
            pl.BlockSpec((bb, _D), lambda i: (i, 0)),
            pl.BlockSpec((bb, _D), lambda i: (i, 0)),
            pl.BlockSpec(w1aT.shape, lambda i: (0, 0)),
            pl.BlockSpec(w1bT.shape, lambda i: (0, 0)),
            pl.BlockSpec((1, 128), lambda i: (0, 0)),
            pl.BlockSpec((128, 64), lambda i: (0, 0)),
            pl.BlockSpec((1, 64), lambda i: (0, 0)),
            pl.BlockSpec((64, 1), lambda i: (0, 0)),
            pl.BlockSpec((1, 1), lambda i: (0, 0)),
        ],
        out_specs=pl.BlockSpec((bb, 1), lambda i: (i, 0)),
        out_shape=jax.ShapeDtypeStruct((_B, 1), jnp.float32),
    )(u_rows, i_rows, w1aT, w1bT, b1.reshape(1, 128), W2.T,
      b2.reshape(1, 64), W3.T, b3.reshape(1, 1))
    return out


# speedup vs baseline: 2.4320x; 1.0036x over previous
import functools

import jax
import jax.numpy as jnp
from jax import lax
from jax.experimental import pallas as pl
from jax.experimental.pallas import tpu as pltpu
from jax.experimental.pallas import tpu_sc as plsc

_B = 16384
_D = 50
_CHUNK = 256


def _gather_body(nc, b_per_w,
                 user_hbm, item_hbm, utab_hbm, itab_hbm, uout_hbm, iout_hbm,
                 uidx_v, iidx_v, urows_v, irows_v, gsem):
    wid = lax.axis_index("s") * nc + lax.axis_index("c")
    base = wid * b_per_w
    pltpu.sync_copy(user_hbm.at[wid], uidx_v)
    pltpu.sync_copy(item_hbm.at[wid], iidx_v)
    lanes = lax.iota(jnp.int32, 16)
    n_chunks = b_per_w // _CHUNK

    for c in range(n_chunks):
        def group(k, _, c=c):
            off = pl.multiple_of(c * _CHUNK + k * 16, 16)
            uvec = uidx_v[pl.ds(off, 16)]
            ivec = iidx_v[pl.ds(off, 16)]
            for j in range(16):
                us = jnp.sum(jnp.where(lanes == j, uvec, 0))
                isc = jnp.sum(jnp.where(lanes == j, ivec, 0))
                row = k * 16 + j
                pltpu.async_copy(utab_hbm.at[pl.ds(us, 1)],
                                 urows_v.at[pl.ds(row, 1)], gsem)
                pltpu.async_copy(itab_hbm.at[pl.ds(isc, 1)],
                                 irows_v.at[pl.ds(row, 1)], gsem)
            return 0

        lax.fori_loop(0, _CHUNK // 16, group, 0)
        pltpu.make_async_copy(uout_hbm.at[pl.ds(0, _CHUNK)], urows_v,
                              gsem).wait()
        pltpu.make_async_copy(uout_hbm.at[pl.ds(0, _CHUNK)], irows_v,
                              gsem).wait()
        pltpu.sync_copy(urows_v, uout_hbm.at[pl.ds(base + c * _CHUNK, _CHUNK)])
        pltpu.sync_copy(irows_v, iout_hbm.at[pl.ds(base + c * _CHUNK, _CHUNK)])


def _mlp_body(u_ref, i_ref, w1a_ref, w1b_ref, b1_ref, w2_ref, b2_ref,
              w3_ref, b3_ref, out_ref):
    u = u_ref[...]
    i = i_ref[...]
    h = jnp.dot(u, w1a_ref[...], preferred_element_type=jnp.float32)
    h = h + jnp.dot(i, w1b_ref[...], preferred_element_type=jnp.float32)
    h = jnp.maximum(h + b1_ref[...], 0.0)
    h = jnp.maximum(
        jnp.dot(h, w2_ref[...], preferred_element_type=jnp.float32)
        + b2_ref[...], 0.0)
    out_ref[...] = (jnp.dot(h, w3_ref[...],
                            preferred_element_type=jnp.float32)
                    + b3_ref[...])


def kernel(user, item, user_table, item_table, W1, b1, W2, b2, W3, b3):
    info = plsc.get_sparse_core_info()
    nc, ns = info.num_cores, info.num_subcores
    nw = nc * ns
    b_per_w = _B // nw

    mesh = plsc.VectorSubcoreMesh(core_axis_name="c", subcore_axis_name="s")
    gather = functools.partial(
        pl.kernel,
        out_type=(jax.ShapeDtypeStruct((_B, _D), jnp.float32),
                  jax.ShapeDtypeStruct((_B, _D), jnp.float32)),
        mesh=mesh,
        scratch_types=[
            pltpu.VMEM((b_per_w,), jnp.int32),
            pltpu.VMEM((b_per_w,), jnp.int32),
            pltpu.VMEM((_CHUNK, _D), jnp.float32),
            pltpu.VMEM((_CHUNK, _D), jnp.float32),
            pltpu.SemaphoreType.DMA,
        ],
        compiler_params=pltpu.CompilerParams(needs_layout_passes=False),
    )(functools.partial(_gather_body, nc, b_per_w))

    u_rows, i_rows = gather(
        user.reshape(nw, b_per_w).astype(jnp.int32),
        item.reshape(nw, b_per_w).astype(jnp.int32),
        user_table, item_table)

    bb = 8192
    grid = _B // bb
    w1aT = W1[:, :_D].T
    w1bT = W1[:, _D:].T
    out = pl.pallas_call(
        _mlp_body,
        grid=(grid,),
        in_specs=[
            pl.BlockSpec((bb, _D), lambda i: (i, 0)),
            pl.BlockSpec((bb, _D), lambda i: (i, 0)),
            pl.BlockSpec(w1aT.shape, lambda i: (0, 0)),
            pl.BlockSpec(w1bT.shape, lambda i: (0, 0)),
            pl.BlockSpec((1, 128), lambda i: (0, 0)),
            pl.BlockSpec((128, 64), lambda i: (0, 0)),
            pl.BlockSpec((1, 64), lambda i: (0, 0)),
            pl.BlockSpec((64, 1), lambda i: (0, 0)),
            pl.BlockSpec((1, 1), lambda i: (0, 0)),
        ],
        out_specs=pl.BlockSpec((bb, 1), lambda i: (i, 0)),
        out_shape=jax.ShapeDtypeStruct((_B, 1), jnp.float32),
    )(u_rows, i_rows, w1aT, w1bT, b1.reshape(1, 128), W2.T,
      b2.reshape(1, 64), W3.T, b3.reshape(1, 1))
    return out
